# baseline (device time: 431095 ns/iter reference)
import jax
import jax.numpy as jnp
from jax import lax
from jax.experimental import pallas as pl
from jax.experimental.pallas import tpu as pltpu

N_DEV = 32
B = 512
D = 512
CH = B // N_DEV


def _mod(v):
    return lax.rem(v + 2 * N_DEV, N_DEV)


def kernel(x, Win0, Wout0, Win1, Wout1, Win2, Wout2):
    b, k_shard = x.shape
    d_out = Wout0.shape[1]

    def body(x_ref, win0_ref, wout0_ref, win1_ref, wout1_ref,
             win2_ref, wout2_ref, out_ref,
             a_ref, r_ref, h_ref,
             rs_send, rs_recv, ag_send, ag_recv, bar):
        p = lax.axis_index("i")
        left = _mod(p - 1)
        right = _mod(p + 1)

        bsem = pltpu.get_barrier_semaphore()
        for nbr in (left, right):
            pl.semaphore_signal(bsem, inc=1, device_id=(nbr,),
                                device_id_type=pl.DeviceIdType.MESH)
        pl.semaphore_wait(bsem, 2)

        wins = [win0_ref, win1_ref, win2_ref]
        wouts = [wout0_ref, wout1_ref, wout2_ref]
        cur = x_ref[:, :]
        for l in range(3):
            a_ref[:, :] = jnp.dot(cur, wins[l][:, :],
                                  preferred_element_type=jnp.float32)

            for h in range(N_DEV - 1):
                c_s = _mod(p - h)
                c_r = _mod(p - h - 1)
                rdma = pltpu.make_async_remote_copy(
                    src_ref=a_ref.at[pl.ds(c_s * CH, CH), :],
                    dst_ref=r_ref.at[pl.ds(c_s * CH, CH), :],
                    send_sem=rs_send.at[h],
                    recv_sem=rs_recv.at[h],
                    device_id=(right,),
                    device_id_type=pl.DeviceIdType.MESH,
                )
                rdma.start()
                rdma.wait()
                a_ref[pl.ds(c_r * CH, CH), :] = (
                    a_ref[pl.ds(c_r * CH, CH), :]
                    + r_ref[pl.ds(c_r * CH, CH), :]
                )

            c_own = _mod(p + 1)
            h_ref[pl.ds(c_own * CH, CH), :] = jnp.maximum(
                a_ref[pl.ds(c_own * CH, CH), :], 0.0)

            for h in range(N_DEV - 1):
                s = _mod(p + 1 - h)
                rdma = pltpu.make_async_remote_copy(
                    src_ref=h_ref.at[pl.ds(s * CH, CH), :],
                    dst_ref=h_ref.at[pl.ds(s * CH, CH), :],
                    send_sem=ag_send.at[h],
                    recv_sem=ag_recv.at[h],
                    device_id=(right,),
                    device_id_type=pl.DeviceIdType.MESH,
                )
                rdma.start()
                rdma.wait()

            cur = jnp.dot(h_ref[:, :], wouts[l][:, :],
                          preferred_element_type=jnp.float32)

            pl.semaphore_signal(bar.at[l, 0], inc=1, device_id=(left,),
                                device_id_type=pl.DeviceIdType.MESH)
            pl.semaphore_signal(bar.at[l, 1], inc=1, device_id=(right,),
                                device_id_type=pl.DeviceIdType.MESH)
            pl.semaphore_wait(bar.at[l, 0], 1)
            pl.semaphore_wait(bar.at[l, 1], 1)

        out_ref[:, :] = cur

    return pl.pallas_call(
        body,
        out_shape=jax.ShapeDtypeStruct((b, d_out), jnp.float32),
        in_specs=[pl.BlockSpec(memory_space=pltpu.VMEM)] * 7,
        out_specs=pl.BlockSpec(memory_space=pltpu.VMEM),
        scratch_shapes=[
            pltpu.VMEM((B, D), jnp.float32),
            pltpu.VMEM((B, D), jnp.float32),
            pltpu.VMEM((B, D), jnp.float32),
            pltpu.SemaphoreType.DMA((N_DEV - 1,)),
            pltpu.SemaphoreType.DMA((N_DEV - 1,)),
            pltpu.SemaphoreType.DMA((N_DEV - 1,)),
            pltpu.SemaphoreType.DMA((N_DEV - 1,)),
            pltpu.SemaphoreType.REGULAR((3, 2)),
        ],
        compiler_params=pltpu.CompilerParams(collective_id=0),
    )(x, Win0, Wout0, Win1, Wout1, Win2, Wout2)


# device time: 127371 ns/iter; 3.3846x vs baseline; 3.3846x over previous
import jax
import jax.numpy as jnp
from jax import lax
from jax.experimental import pallas as pl
from jax.experimental.pallas import tpu as pltpu

N_DEV = 32
B = 512
D = 512
MESH = pl.DeviceIdType.MESH


def kernel(x, Win0, Wout0, Win1, Wout1, Win2, Wout2):
    b, k_shard = x.shape
    d_out = Wout0.shape[1]

    def body(x_ref, win0_ref, wout0_ref, win1_ref, wout1_ref,
             win2_ref, wout2_ref, out_ref,
             a_ref, rx_ref, ry_ref, rz_ref, h_ref,
             x_send, x_recv, y_send, y_recv, z_send, z_recv,
             agx_send, agx_recv, agy_send, agy_recv, agz_send, agz_recv,
             bar):
        p = lax.axis_index("i")
        t = lax.rem(p, 2)
        zc = lax.div(p, 8)
        q = lax.rem(p, 8)
        yc = lax.div(q, 2)
        yb = lax.rem(yc, 2)
        xc = t + yb - 2 * t * yb
        base8 = lax.rem(p, 8)

        xpeer = p + 1 - 2 * t

        def ypos(yp):
            ypb = lax.rem(yp, 2)
            tt = xc + ypb - 2 * xc * ypb
            return 8 * zc + 2 * yp + tt

        def zpos(zp):
            return base8 + 8 * zp

        x_off = xc * 256
        y_off = x_off + yc * 64
        z_off = y_off + zc * 16

        bsem = pltpu.get_barrier_semaphore()
        pl.semaphore_signal(bsem, inc=1, device_id=(xpeer,),
                            device_id_type=MESH)
        for d in range(1, 4):
            pl.semaphore_signal(bsem, inc=1,
                                device_id=(ypos(lax.rem(yc + d, 4)),),
                                device_id_type=MESH)
            pl.semaphore_signal(bsem, inc=1,
                                device_id=(zpos(lax.rem(zc + d, 4)),),
                                device_id_type=MESH)
        pl.semaphore_wait(bsem, 7)

        wins = [win0_ref, win1_ref, win2_ref]
        wouts = [wout0_ref, wout1_ref, wout2_ref]
        cur = x_ref[:, :]
        for l in range(3):
            a_ref[:, :] = jnp.dot(cur, wins[l][:, :],
                                  preferred_element_type=jnp.float32)

            rs_x = pltpu.make_async_remote_copy(
                src_ref=a_ref.at[pl.ds((1 - xc) * 256, 256), :],
                dst_ref=rx_ref,
                send_sem=x_send.at[0], recv_sem=x_recv.at[0],
                device_id=(xpeer,), device_id_type=MESH,
            )
            rs_x.start()
            rs_x.wait()
            a_ref[pl.ds(x_off, 256), :] = (
                a_ref[pl.ds(x_off, 256), :] + rx_ref[:, :])

            for d in range(1, 4):
                yp = lax.rem(yc + d, 4)
                pltpu.make_async_remote_copy(
                    src_ref=a_ref.at[pl.ds(x_off + yp * 64, 64), :],
                    dst_ref=ry_ref.at[pl.ds((4 - d) * 64 - 64, 64), :],
                    send_sem=y_send.at[d], recv_sem=y_recv.at[d],
                    device_id=(ypos(yp),), device_id_type=MESH,
                ).start()
            for d in range(1, 4):
                desc = pltpu.make_async_remote_copy(
                    src_ref=a_ref.at[pl.ds(x_off, 64), :],
                    dst_ref=ry_ref.at[pl.ds((4 - d) * 64 - 64, 64), :],
                    send_sem=y_send.at[d], recv_sem=y_recv.at[d],
                    device_id=(p,), device_id_type=MESH,
                )
                desc.wait_recv()
                desc.wait_send()
            own_y = a_ref[pl.ds(y_off, 64), :]
            acc_y = (own_y + ry_ref[pl.ds(0, 64), :]
                     + ry_ref[pl.ds(64, 64), :] + ry_ref[pl.ds(128, 64), :])
            a_ref[pl.ds(y_off, 64), :] = acc_y

            for d in range(1, 4):
                zp = lax.rem(zc + d, 4)
                pltpu.make_async_remote_copy(
                    src_ref=a_ref.at[pl.ds(y_off + zp * 16, 16), :],
                    dst_ref=rz_ref.at[pl.ds((4 - d) * 16 - 16, 16), :],
                    send_sem=z_send.at[d], recv_sem=z_recv.at[d],
                    device_id=(zpos(zp),), device_id_type=MESH,
                ).start()
            for d in range(1, 4):
                desc = pltpu.make_async_remote_copy(
                    src_ref=a_ref.at[pl.ds(y_off, 16), :],
                    dst_ref=rz_ref.at[pl.ds((4 - d) * 16 - 16, 16), :],
                    send_sem=z_send.at[d], recv_sem=z_recv.at[d],
                    device_id=(p,), device_id_type=MESH,
                )
                desc.wait_recv()
                desc.wait_send()
            acc_z = (a_ref[pl.ds(z_off, 16), :] + rz_ref[pl.ds(0, 16), :]
                     + rz_ref[pl.ds(16, 16), :] + rz_ref[pl.ds(32, 16), :])

            h_ref[pl.ds(z_off, 16), :] = jnp.maximum(acc_z, 0.0)

            for d in range(1, 4):
                zp = lax.rem(zc + d, 4)
                pltpu.make_async_remote_copy(
                    src_ref=h_ref.at[pl.ds(z_off, 16), :],
                    dst_ref=h_ref.at[pl.ds(z_off, 16), :],
                    send_sem=agz_send.at[d], recv_sem=agz_recv.at[d],
                    device_id=(zpos(zp),), device_id_type=MESH,
                ).start()
            for d in range(1, 4):
                zs = lax.rem(zc + 4 - d, 4)
                desc = pltpu.make_async_remote_copy(
                    src_ref=h_ref.at[pl.ds(z_off, 16), :],
                    dst_ref=h_ref.at[pl.ds(y_off + zs * 16, 16), :],
                    send_sem=agz_send.at[d], recv_sem=agz_recv.at[d],
                    device_id=(p,), device_id_type=MESH,
                )
                desc.wait_recv()
                desc.wait_send()

            for d in range(1, 4):
                yp = lax.rem(yc + d, 4)
                pltpu.make_async_remote_copy(
                    src_ref=h_ref.at[pl.ds(y_off, 64), :],
                    dst_ref=h_ref.at[pl.ds(y_off, 64), :],
                    send_sem=agy_send.at[d], recv_sem=agy_recv.at[d],
                    device_id=(ypos(yp),), device_id_type=MESH,
                ).start()
            for d in range(1, 4):
                ys = lax.rem(yc + 4 - d, 4)
                desc = pltpu.make_async_remote_copy(
                    src_ref=h_ref.at[pl.ds(y_off, 64), :],
                    dst_ref=h_ref.at[pl.ds(x_off + ys * 64, 64), :],
                    send_sem=agy_send.at[d], recv_sem=agy_recv.at[d],
                    device_id=(p,), device_id_type=MESH,
                )
                desc.wait_recv()
                desc.wait_send()

            ag_x = pltpu.make_async_remote_copy(
                src_ref=h_ref.at[pl.ds(x_off, 256), :],
                dst_ref=h_ref.at[pl.ds(x_off, 256), :],
                send_sem=agx_send.at[0], recv_sem=agx_recv.at[0],
                device_id=(xpeer,), device_id_type=MESH,
            )
            ag_x.start()
            ag_x.wait_send()
            desc = pltpu.make_async_remote_copy(
                src_ref=h_ref.at[pl.ds(x_off, 256), :],
                dst_ref=h_ref.at[pl.ds((1 - xc) * 256, 256), :],
                send_sem=agx_send.at[0], recv_sem=agx_recv.at[0],
                device_id=(p,), device_id_type=MESH,
            )
            desc.wait_recv()

            cur = jnp.dot(h_ref[:, :], wouts[l][:, :],
                          preferred_element_type=jnp.float32)

            pl.semaphore_signal(bar.at[l, 0], inc=1, device_id=(xpeer,),
                                device_id_type=MESH)
            for d in range(1, 4):
                pl.semaphore_signal(bar.at[l, 1], inc=1,
                                    device_id=(ypos(lax.rem(yc + d, 4)),),
                                    device_id_type=MESH)
                pl.semaphore_signal(bar.at[l, 2], inc=1,
                                    device_id=(zpos(lax.rem(zc + d, 4)),),
                                    device_id_type=MESH)
            pl.semaphore_wait(bar.at[l, 0], 1)
            pl.semaphore_wait(bar.at[l, 1], 3)
            pl.semaphore_wait(bar.at[l, 2], 3)

        out_ref[:, :] = cur

    return pl.pallas_call(
        body,
        out_shape=jax.ShapeDtypeStruct((b, d_out), jnp.float32),
        in_specs=[pl.BlockSpec(memory_space=pltpu.VMEM)] * 7,
        out_specs=pl.BlockSpec(memory_space=pltpu.VMEM),
        scratch_shapes=[
            pltpu.VMEM((B, D), jnp.float32),
            pltpu.VMEM((256, D), jnp.float32),
            pltpu.VMEM((192, D), jnp.float32),
            pltpu.VMEM((48, D), jnp.float32),
            pltpu.VMEM((B, D), jnp.float32),
            pltpu.SemaphoreType.DMA((1,)),
            pltpu.SemaphoreType.DMA((1,)),
            pltpu.SemaphoreType.DMA((4,)),
            pltpu.SemaphoreType.DMA((4,)),
            pltpu.SemaphoreType.DMA((4,)),
            pltpu.SemaphoreType.DMA((4,)),
            pltpu.SemaphoreType.DMA((1,)),
            pltpu.SemaphoreType.DMA((1,)),
            pltpu.SemaphoreType.DMA((4,)),
            pltpu.SemaphoreType.DMA((4,)),
            pltpu.SemaphoreType.DMA((4,)),
            pltpu.SemaphoreType.DMA((4,)),
            pltpu.SemaphoreType.REGULAR((3, 3)),
        ],
        compiler_params=pltpu.CompilerParams(collective_id=0),
    )(x, Win0, Wout0, Win1, Wout1, Win2, Wout2)


# device time: 118630 ns/iter; 3.6339x vs baseline; 1.0737x over previous
import jax
import jax.numpy as jnp
from jax import lax
from jax.experimental import pallas as pl
from jax.experimental.pallas import tpu as pltpu

N_DEV = 32
B = 512
D = 512
MESH = pl.DeviceIdType.MESH


def kernel(x, Win0, Wout0, Win1, Wout1, Win2, Wout2):
    b, k_shard = x.shape
    d_out = Wout0.shape[1]

    def body(x_ref, win0_ref, wout0_ref, win1_ref, wout1_ref,
             win2_ref, wout2_ref, out_ref,
             a_ref, rx_ref, ry_ref, rz_ref, h_ref,
             x_send, x_recv, y_send, y_recv, z_send, z_recv,
             agx_send, agx_recv, agy_send, agy_recv, agz_send, agz_recv):
        p = lax.axis_index("i")
        t = lax.rem(p, 2)
        zc = lax.div(p, 8)
        q = lax.rem(p, 8)
        yc = lax.div(q, 2)
        yb = lax.rem(yc, 2)
        xc = t + yb - 2 * t * yb
        base8 = lax.rem(p, 8)

        xpeer = p + 1 - 2 * t

        def ypos(yp):
            ypb = lax.rem(yp, 2)
            tt = xc + ypb - 2 * xc * ypb
            return 8 * zc + 2 * yp + tt

        def zpos(zp):
            return base8 + 8 * zp

        x_off = xc * 256
        y_off = x_off + yc * 64
        z_off = y_off + zc * 16

        bsem = pltpu.get_barrier_semaphore()
        pl.semaphore_signal(bsem, inc=1, device_id=(xpeer,),
                            device_id_type=MESH)
        for d in range(1, 4):
            pl.semaphore_signal(bsem, inc=1,
                                device_id=(ypos(lax.rem(yc + d, 4)),),
                                device_id_type=MESH)
            pl.semaphore_signal(bsem, inc=1,
                                device_id=(zpos(lax.rem(zc + d, 4)),),
                                device_id_type=MESH)
        pl.semaphore_wait(bsem, 7)

        wins = [win0_ref, win1_ref, win2_ref]
        wouts = [wout0_ref, wout1_ref, wout2_ref]
        cur = x_ref[:, :]
        for l in range(3):
            a_ref[:, :] = jnp.dot(cur, wins[l][:, :],
                                  preferred_element_type=jnp.float32)

            rs_x = pltpu.make_async_remote_copy(
                src_ref=a_ref.at[pl.ds((1 - xc) * 256, 256), :],
                dst_ref=rx_ref,
                send_sem=x_send.at[0], recv_sem=x_recv.at[0],
                device_id=(xpeer,), device_id_type=MESH,
            )
            rs_x.start()
            rs_x.wait()
            a_ref[pl.ds(x_off, 256), :] = (
                a_ref[pl.ds(x_off, 256), :] + rx_ref[:, :])

            for d in range(1, 4):
                yp = lax.rem(yc + d, 4)
                pltpu.make_async_remote_copy(
                    src_ref=a_ref.at[pl.ds(x_off + yp * 64, 64), :],
                    dst_ref=ry_ref.at[pl.ds((4 - d) * 64 - 64, 64), :],
                    send_sem=y_send.at[d], recv_sem=y_recv.at[d],
                    device_id=(ypos(yp),), device_id_type=MESH,
                ).start()
            for d in range(1, 4):
                desc = pltpu.make_async_remote_copy(
                    src_ref=a_ref.at[pl.ds(x_off, 64), :],
                    dst_ref=ry_ref.at[pl.ds((4 - d) * 64 - 64, 64), :],
                    send_sem=y_send.at[d], recv_sem=y_recv.at[d],
                    device_id=(p,), device_id_type=MESH,
                )
                desc.wait_recv()
                desc.wait_send()
            own_y = a_ref[pl.ds(y_off, 64), :]
            acc_y = (own_y + ry_ref[pl.ds(0, 64), :]
                     + ry_ref[pl.ds(64, 64), :] + ry_ref[pl.ds(128, 64), :])
            a_ref[pl.ds(y_off, 64), :] = acc_y

            for d in range(1, 4):
                zp = lax.rem(zc + d, 4)
                pltpu.make_async_remote_copy(
                    src_ref=a_ref.at[pl.ds(y_off + zp * 16, 16), :],
                    dst_ref=rz_ref.at[pl.ds((4 - d) * 16 - 16, 16), :],
                    send_sem=z_send.at[d], recv_sem=z_recv.at[d],
                    device_id=(zpos(zp),), device_id_type=MESH,
                ).start()
            for d in range(1, 4):
                desc = pltpu.make_async_remote_copy(
                    src_ref=a_ref.at[pl.ds(y_off, 16), :],
                    dst_ref=rz_ref.at[pl.ds((4 - d) * 16 - 16, 16), :],
                    send_sem=z_send.at[d], recv_sem=z_recv.at[d],
                    device_id=(p,), device_id_type=MESH,
                )
                desc.wait_recv()
                desc.wait_send()
            acc_z = (a_ref[pl.ds(z_off, 16), :] + rz_ref[pl.ds(0, 16), :]
                     + rz_ref[pl.ds(16, 16), :] + rz_ref[pl.ds(32, 16), :])

            h_ref[pl.ds(z_off, 16), :] = jnp.maximum(acc_z, 0.0)

            for d in range(1, 4):
                zp = lax.rem(zc + d, 4)
                pltpu.make_async_remote_copy(
                    src_ref=h_ref.at[pl.ds(z_off, 16), :],
                    dst_ref=h_ref.at[pl.ds(z_off, 16), :],
                    send_sem=agz_send.at[d], recv_sem=agz_recv.at[d],
                    device_id=(zpos(zp),), device_id_type=MESH,
                ).start()
            for d in range(1, 4):
                zs = lax.rem(zc + 4 - d, 4)
                desc = pltpu.make_async_remote_copy(
                    src_ref=h_ref.at[pl.ds(z_off, 16), :],
                    dst_ref=h_ref.at[pl.ds(y_off + zs * 16, 16), :],
                    send_sem=agz_send.at[d], recv_sem=agz_recv.at[d],
                    device_id=(p,), device_id_type=MESH,
                )
                desc.wait_recv()
                desc.wait_send()

            for d in range(1, 4):
                yp = lax.rem(yc + d, 4)
                pltpu.make_async_remote_copy(
                    src_ref=h_ref.at[pl.ds(y_off, 64), :],
                    dst_ref=h_ref.at[pl.ds(y_off, 64), :],
                    send_sem=agy_send.at[d], recv_sem=agy_recv.at[d],
                    device_id=(ypos(yp),), device_id_type=MESH,
                ).start()
            for d in range(1, 4):
                ys = lax.rem(yc + 4 - d, 4)
                desc = pltpu.make_async_remote_copy(
                    src_ref=h_ref.at[pl.ds(y_off, 64), :],
                    dst_ref=h_ref.at[pl.ds(x_off + ys * 64, 64), :],
                    send_sem=agy_send.at[d], recv_sem=agy_recv.at[d],
                    device_id=(p,), device_id_type=MESH,
                )
                desc.wait_recv()
                desc.wait_send()

            ag_x = pltpu.make_async_remote_copy(
                src_ref=h_ref.at[pl.ds(x_off, 256), :],
                dst_ref=h_ref.at[pl.ds(x_off, 256), :],
                send_sem=agx_send.at[0], recv_sem=agx_recv.at[0],
                device_id=(xpeer,), device_id_type=MESH,
            )
            ag_x.start()
            ag_x.wait_send()
            desc = pltpu.make_async_remote_copy(
                src_ref=h_ref.at[pl.ds(x_off, 256), :],
                dst_ref=h_ref.at[pl.ds((1 - xc) * 256, 256), :],
                send_sem=agx_send.at[0], recv_sem=agx_recv.at[0],
                device_id=(p,), device_id_type=MESH,
            )
            desc.wait_recv()

            cur = jnp.dot(h_ref[:, :], wouts[l][:, :],
                          preferred_element_type=jnp.float32)


        out_ref[:, :] = cur

    return pl.pallas_call(
        body,
        out_shape=jax.ShapeDtypeStruct((b, d_out), jnp.float32),
        in_specs=[pl.BlockSpec(memory_space=pltpu.VMEM)] * 7,
        out_specs=pl.BlockSpec(memory_space=pltpu.VMEM),
        scratch_shapes=[
            pltpu.VMEM((B, D), jnp.float32),
            pltpu.VMEM((256, D), jnp.float32),
            pltpu.VMEM((192, D), jnp.float32),
            pltpu.VMEM((48, D), jnp.float32),
            pltpu.VMEM((B, D), jnp.float32),
            pltpu.SemaphoreType.DMA((1,)),
            pltpu.SemaphoreType.DMA((1,)),
            pltpu.SemaphoreType.DMA((4,)),
            pltpu.SemaphoreType.DMA((4,)),
            pltpu.SemaphoreType.DMA((4,)),
            pltpu.SemaphoreType.DMA((4,)),
            pltpu.SemaphoreType.DMA((1,)),
            pltpu.SemaphoreType.DMA((1,)),
            pltpu.SemaphoreType.DMA((4,)),
            pltpu.SemaphoreType.DMA((4,)),
            pltpu.SemaphoreType.DMA((4,)),
            pltpu.SemaphoreType.DMA((4,)),
        ],
        compiler_params=pltpu.CompilerParams(collective_id=0),
    )(x, Win0, Wout0, Win1, Wout1, Win2, Wout2)


# device time: 95762 ns/iter; 4.5017x vs baseline; 1.2388x over previous
import jax
import jax.numpy as jnp
from jax import lax
from jax.experimental import pallas as pl
from jax.experimental.pallas import tpu as pltpu

N_DEV = 32
B = 512
D = 512
HC = 256
MESH = pl.DeviceIdType.MESH


def kernel(x, Win0, Wout0, Win1, Wout1, Win2, Wout2):
    b, k_shard = x.shape
    d_out = Wout0.shape[1]

    def body(x_ref, win0_ref, wout0_ref, win1_ref, wout1_ref,
             win2_ref, wout2_ref, out_ref,
             a_ref, h_ref,
             rxa_ref, rya_ref, rza_ref, ryb_ref, rzb_ref, rxb_ref,
             xa_send, xa_recv, ya_send, ya_recv, za_send, za_recv,
             gxa_send, gxa_recv, gya_send, gya_recv, gza_send, gza_recv,
             xb_send, xb_recv, yb_send, yb_recv, zb_send, zb_recv,
             gxb_send, gxb_recv, gyb_send, gyb_recv, gzb_send, gzb_recv):
        p = lax.axis_index("i")
        t = lax.rem(p, 2)
        zc = lax.div(p, 8)
        q = lax.rem(p, 8)
        yc = lax.div(q, 2)
        yb = lax.rem(yc, 2)
        xc = t + yb - 2 * t * yb
        base8 = lax.rem(p, 8)

        xpeer = p + 1 - 2 * t

        def ypos(yp):
            ypb = lax.rem(yp, 2)
            tt = xc + ypb - 2 * xc * ypb
            return 8 * zc + 2 * yp + tt

        def zpos(zp):
            return base8 + 8 * zp

        CA = pl.ds(0, HC)
        CB = pl.ds(HC, HC)

        xa_off = xc * 256
        ya_off = xa_off + yc * 64
        za_off = ya_off + zc * 16
        yb_off = yc * 128
        zb_off = yb_off + zc * 32
        xb_off = zb_off + xc * 16

        bsem = pltpu.get_barrier_semaphore()
        pl.semaphore_signal(bsem, inc=1, device_id=(xpeer,),
                            device_id_type=MESH)
        for d in range(1, 4):
            pl.semaphore_signal(bsem, inc=1,
                                device_id=(ypos(lax.rem(yc + d, 4)),),
                                device_id_type=MESH)
            pl.semaphore_signal(bsem, inc=1,
                                device_id=(zpos(lax.rem(zc + d, 4)),),
                                device_id_type=MESH)
        pl.semaphore_wait(bsem, 7)

        wins = [win0_ref, win1_ref, win2_ref]
        wouts = [wout0_ref, wout1_ref, wout2_ref]
        cur = x_ref[:, :]
        for l in range(3):
            a_ref[:, :] = jnp.dot(cur, wins[l][:, :],
                                  preferred_element_type=jnp.float32)

            rs_xa = pltpu.make_async_remote_copy(
                src_ref=a_ref.at[pl.ds((1 - xc) * 256, 256), CA],
                dst_ref=rxa_ref,
                send_sem=xa_send.at[0], recv_sem=xa_recv.at[0],
                device_id=(xpeer,), device_id_type=MESH,
            )
            rs_xa.start()

            for d in range(1, 4):
                yp = lax.rem(yc + d, 4)
                pltpu.make_async_remote_copy(
                    src_ref=a_ref.at[pl.ds(yp * 128, 128), CB],
                    dst_ref=ryb_ref.at[pl.ds((3 - d) * 128, 128), :],
                    send_sem=yb_send.at[d], recv_sem=yb_recv.at[d],
                    device_id=(ypos(yp),), device_id_type=MESH,
                ).start()

            rs_xa.wait()
            a_ref[pl.ds(xa_off, 256), CA] = (
                a_ref[pl.ds(xa_off, 256), CA] + rxa_ref[:, :])
            for d in range(1, 4):
                yp = lax.rem(yc + d, 4)
                pltpu.make_async_remote_copy(
                    src_ref=a_ref.at[pl.ds(xa_off + yp * 64, 64), CA],
                    dst_ref=rya_ref.at[pl.ds((3 - d) * 64, 64), :],
                    send_sem=ya_send.at[d], recv_sem=ya_recv.at[d],
                    device_id=(ypos(yp),), device_id_type=MESH,
                ).start()

            for d in range(1, 4):
                desc = pltpu.make_async_remote_copy(
                    src_ref=a_ref.at[pl.ds(0, 128), CB],
                    dst_ref=ryb_ref.at[pl.ds((3 - d) * 128, 128), :],
                    send_sem=yb_send.at[d], recv_sem=yb_recv.at[d],
                    device_id=(p,), device_id_type=MESH,
                )
                desc.wait_recv()
                desc.wait_send()
            a_ref[pl.ds(yb_off, 128), CB] = (
                a_ref[pl.ds(yb_off, 128), CB] + ryb_ref[pl.ds(0, 128), :]
                + ryb_ref[pl.ds(128, 128), :] + ryb_ref[pl.ds(256, 128), :])
            for d in range(1, 4):
                zp = lax.rem(zc + d, 4)
                pltpu.make_async_remote_copy(
                    src_ref=a_ref.at[pl.ds(yb_off + zp * 32, 32), CB],
                    dst_ref=rzb_ref.at[pl.ds((3 - d) * 32, 32), :],
                    send_sem=zb_send.at[d], recv_sem=zb_recv.at[d],
                    device_id=(zpos(zp),), device_id_type=MESH,
                ).start()

            for d in range(1, 4):
                desc = pltpu.make_async_remote_copy(
                    src_ref=a_ref.at[pl.ds(0, 64), CA],
                    dst_ref=rya_ref.at[pl.ds((3 - d) * 64, 64), :],
                    send_sem=ya_send.at[d], recv_sem=ya_recv.at[d],
                    device_id=(p,), device_id_type=MESH,
                )
                desc.wait_recv()
                desc.wait_send()
            a_ref[pl.ds(ya_off, 64), CA] = (
                a_ref[pl.ds(ya_off, 64), CA] + rya_ref[pl.ds(0, 64), :]
                + rya_ref[pl.ds(64, 64), :] + rya_ref[pl.ds(128, 64), :])
            for d in range(1, 4):
                zp = lax.rem(zc + d, 4)
                pltpu.make_async_remote_copy(
                    src_ref=a_ref.at[pl.ds(ya_off + zp * 16, 16), CA],
                    dst_ref=rza_ref.at[pl.ds((3 - d) * 16, 16), :],
                    send_sem=za_send.at[d], recv_sem=za_recv.at[d],
                    device_id=(zpos(zp),), device_id_type=MESH,
                ).start()

            for d in range(1, 4):
                desc = pltpu.make_async_remote_copy(
                    src_ref=a_ref.at[pl.ds(0, 32), CB],
                    dst_ref=rzb_ref.at[pl.ds((3 - d) * 32, 32), :],
                    send_sem=zb_send.at[d], recv_sem=zb_recv.at[d],
                    device_id=(p,), device_id_type=MESH,
                )
                desc.wait_recv()
                desc.wait_send()
            a_ref[pl.ds(zb_off, 32), CB] = (
                a_ref[pl.ds(zb_off, 32), CB] + rzb_ref[pl.ds(0, 32), :]
                + rzb_ref[pl.ds(32, 32), :] + rzb_ref[pl.ds(64, 32), :])
            rs_xb = pltpu.make_async_remote_copy(
                src_ref=a_ref.at[pl.ds(zb_off + (1 - xc) * 16, 16), CB],
                dst_ref=rxb_ref,
                send_sem=xb_send.at[0], recv_sem=xb_recv.at[0],
                device_id=(xpeer,), device_id_type=MESH,
            )
            rs_xb.start()

            for d in range(1, 4):
                desc = pltpu.make_async_remote_copy(
                    src_ref=a_ref.at[pl.ds(0, 16), CA],
                    dst_ref=rza_ref.at[pl.ds((3 - d) * 16, 16), :],
                    send_sem=za_send.at[d], recv_sem=za_recv.at[d],
                    device_id=(p,), device_id_type=MESH,
                )
                desc.wait_recv()
                desc.wait_send()
            acc_za = (a_ref[pl.ds(za_off, 16), CA] + rza_ref[pl.ds(0, 16), :]
                      + rza_ref[pl.ds(16, 16), :] + rza_ref[pl.ds(32, 16), :])
            h_ref[pl.ds(za_off, 16), CA] = jnp.maximum(acc_za, 0.0)
            for d in range(1, 4):
                zp = lax.rem(zc + d, 4)
                pltpu.make_async_remote_copy(
                    src_ref=h_ref.at[pl.ds(za_off, 16), CA],
                    dst_ref=h_ref.at[pl.ds(za_off, 16), CA],
                    send_sem=gza_send.at[d], recv_sem=gza_recv.at[d],
                    device_id=(zpos(zp),), device_id_type=MESH,
                ).start()

            rs_xb.wait()
            acc_xb = a_ref[pl.ds(xb_off, 16), CB] + rxb_ref[:, :]
            h_ref[pl.ds(xb_off, 16), CB] = jnp.maximum(acc_xb, 0.0)
            ag_xb = pltpu.make_async_remote_copy(
                src_ref=h_ref.at[pl.ds(xb_off, 16), CB],
                dst_ref=h_ref.at[pl.ds(xb_off, 16), CB],
                send_sem=gxb_send.at[0], recv_sem=gxb_recv.at[0],
                device_id=(xpeer,), device_id_type=MESH,
            )
            ag_xb.start()

            for d in range(1, 4):
                zs = lax.rem(zc + 4 - d, 4)
                desc = pltpu.make_async_remote_copy(
                    src_ref=h_ref.at[pl.ds(za_off, 16), CA],
                    dst_ref=h_ref.at[pl.ds(ya_off + zs * 16, 16), CA],
                    send_sem=gza_send.at[d], recv_sem=gza_recv.at[d],
                    device_id=(p,), device_id_type=MESH,
                )
                desc.wait_recv()
                desc.wait_send()
            for d in range(1, 4):
                yp = lax.rem(yc + d, 4)
                pltpu.make_async_remote_copy(
                    src_ref=h_ref.at[pl.ds(ya_off, 64), CA],
                    dst_ref=h_ref.at[pl.ds(ya_off, 64), CA],
                    send_sem=gya_send.at[d], recv_sem=gya_recv.at[d],
                    device_id=(ypos(yp),), device_id_type=MESH,
                ).start()

            ag_xb.wait_send()
            desc = pltpu.make_async_remote_copy(
                src_ref=h_ref.at[pl.ds(xb_off, 16), CB],
                dst_ref=h_ref.at[pl.ds(zb_off + (1 - xc) * 16, 16), CB],
                send_sem=gxb_send.at[0], recv_sem=gxb_recv.at[0],
                device_id=(p,), device_id_type=MESH,
            )
            desc.wait_recv()
            for d in range(1, 4):
                zp = lax.rem(zc + d, 4)
                pltpu.make_async_remote_copy(
                    src_ref=h_ref.at[pl.ds(zb_off, 32), CB],
                    dst_ref=h_ref.at[pl.ds(zb_off, 32), CB],
                    send_sem=gzb_send.at[d], recv_sem=gzb_recv.at[d],
                    device_id=(zpos(zp),), device_id_type=MESH,
                ).start()

            for d in range(1, 4):
                ys = lax.rem(yc + 4 - d, 4)
                desc = pltpu.make_async_remote_copy(
                    src_ref=h_ref.at[pl.ds(ya_off, 64), CA],
                    dst_ref=h_ref.at[pl.ds(xa_off + ys * 64, 64), CA],
                    send_sem=gya_send.at[d], recv_sem=gya_recv.at[d],
                    device_id=(p,), device_id_type=MESH,
                )
                desc.wait_recv()
                desc.wait_send()
            ag_xa = pltpu.make_async_remote_copy(
                src_ref=h_ref.at[pl.ds(xa_off, 256), CA],
                dst_ref=h_ref.at[pl.ds(xa_off, 256), CA],
                send_sem=gxa_send.at[0], recv_sem=gxa_recv.at[0],
                device_id=(xpeer,), device_id_type=MESH,
            )
            ag_xa.start()

            for d in range(1, 4):
                zs = lax.rem(zc + 4 - d, 4)
                desc = pltpu.make_async_remote_copy(
                    src_ref=h_ref.at[pl.ds(zb_off, 32), CB],
                    dst_ref=h_ref.at[pl.ds(yb_off + zs * 32, 32), CB],
                    send_sem=gzb_send.at[d], recv_sem=gzb_recv.at[d],
                    device_id=(p,), device_id_type=MESH,
                )
                desc.wait_recv()
                desc.wait_send()
            for d in range(1, 4):
                yp = lax.rem(yc + d, 4)
                pltpu.make_async_remote_copy(
                    src_ref=h_ref.at[pl.ds(yb_off, 128), CB],
                    dst_ref=h_ref.at[pl.ds(yb_off, 128), CB],
                    send_sem=gyb_send.at[d], recv_sem=gyb_recv.at[d],
                    device_id=(ypos(yp),), device_id_type=MESH,
                ).start()

            ag_xa.wait_send()
            desc = pltpu.make_async_remote_copy(
                src_ref=h_ref.at[pl.ds(xa_off, 256), CA],
                dst_ref=h_ref.at[pl.ds((1 - xc) * 256, 256), CA],
                send_sem=gxa_send.at[0], recv_sem=gxa_recv.at[0],
                device_id=(p,), device_id_type=MESH,
            )
            desc.wait_recv()

            for d in range(1, 4):
                ys = lax.rem(yc + 4 - d, 4)
                desc = pltpu.make_async_remote_copy(
                    src_ref=h_ref.at[pl.ds(yb_off, 128), CB],
                    dst_ref=h_ref.at[pl.ds(ys * 128, 128), CB],
                    send_sem=gyb_send.at[d], recv_sem=gyb_recv.at[d],
                    device_id=(p,), device_id_type=MESH,
                )
                desc.wait_recv()
                desc.wait_send()

            cur = jnp.dot(h_ref[:, :], wouts[l][:, :],
                          preferred_element_type=jnp.float32)

        out_ref[:, :] = cur

    return pl.pallas_call(
        body,
        out_shape=jax.ShapeDtypeStruct((b, d_out), jnp.float32),
        in_specs=[pl.BlockSpec(memory_space=pltpu.VMEM)] * 7,
        out_specs=pl.BlockSpec(memory_space=pltpu.VMEM),
        scratch_shapes=[
            pltpu.VMEM((B, D), jnp.float32),
            pltpu.VMEM((B, D), jnp.float32),
            pltpu.VMEM((256, HC), jnp.float32),
            pltpu.VMEM((192, HC), jnp.float32),
            pltpu.VMEM((48, HC), jnp.float32),
            pltpu.VMEM((384, HC), jnp.float32),
            pltpu.VMEM((96, HC), jnp.float32),
            pltpu.VMEM((16, HC), jnp.float32),
            pltpu.SemaphoreType.DMA((1,)),
            pltpu.SemaphoreType.DMA((1,)),
            pltpu.SemaphoreType.DMA((4,)),
            pltpu.SemaphoreType.DMA((4,)),
            pltpu.SemaphoreType.DMA((4,)),
            pltpu.SemaphoreType.DMA((4,)),
            pltpu.SemaphoreType.DMA((1,)),
            pltpu.SemaphoreType.DMA((1,)),
            pltpu.SemaphoreType.DMA((4,)),
            pltpu.SemaphoreType.DMA((4,)),
            pltpu.SemaphoreType.DMA((4,)),
            pltpu.SemaphoreType.DMA((4,)),
            pltpu.SemaphoreType.DMA((1,)),
            pltpu.SemaphoreType.DMA((1,)),
            pltpu.SemaphoreType.DMA((4,)),
            pltpu.SemaphoreType.DMA((4,)),
            pltpu.SemaphoreType.DMA((4,)),
            pltpu.SemaphoreType.DMA((4,)),
            pltpu.SemaphoreType.DMA((1,)),
            pltpu.SemaphoreType.DMA((1,)),
            pltpu.SemaphoreType.DMA((4,)),
            pltpu.SemaphoreType.DMA((4,)),
            pltpu.SemaphoreType.DMA((4,)),
            pltpu.SemaphoreType.DMA((4,)),
        ],
        compiler_params=pltpu.CompilerParams(collective_id=0),
    )(x, Win0, Wout0, Win1, Wout1, Win2, Wout2)


# device time: 91333 ns/iter; 4.7200x vs baseline; 1.0485x over previous
import jax
import jax.numpy as jnp
from jax import lax
from jax.experimental import pallas as pl
from jax.experimental.pallas import tpu as pltpu

N_DEV = 32
B = 512
D = 512
HC = 256
MESH = pl.DeviceIdType.MESH


def kernel(x, Win0, Wout0, Win1, Wout1, Win2, Wout2):
    b, k_shard = x.shape
    d_out = Wout0.shape[1]

    def body(x_ref, win0_ref, wout0_ref, win1_ref, wout1_ref,
             win2_ref, wout2_ref, out_ref,
             a_ref, h_ref,
             rxa_ref, rya_ref, rza_ref, ryb_ref, rzb_ref, rxb_ref,
             xa_send, xa_recv, ya_send, ya_recv, za_send, za_recv,
             gxa_send, gxa_recv, gya_send, gya_recv,
             xb_send, xb_recv, yb_send, yb_recv, zb_send, zb_recv,
             gyb_send, gyb_recv, gzb_send, gzb_recv):
        p = lax.axis_index("i")
        t = lax.rem(p, 2)
        zc = lax.div(p, 8)
        q = lax.rem(p, 8)
        yc = lax.div(q, 2)
        yb = lax.rem(yc, 2)
        xc = t + yb - 2 * t * yb
        base8 = lax.rem(p, 8)

        xpeer = p + 1 - 2 * t

        def ypos(yp):
            ypb = lax.rem(yp, 2)
            tt = xc + ypb - 2 * xc * ypb
            return 8 * zc + 2 * yp + tt

        def zpos(zp):
            return base8 + 8 * zp

        CA = pl.ds(0, HC)
        CB = pl.ds(HC, HC)

        xa_off = xc * 256
        ya_off = xa_off + yc * 64
        za_off = ya_off + zc * 16
        yb_off = yc * 128
        zb_off = yb_off + zc * 32
        xb_off = zb_off + xc * 16

        bsem = pltpu.get_barrier_semaphore()
        pl.semaphore_signal(bsem, inc=1, device_id=(xpeer,),
                            device_id_type=MESH)
        for d in range(1, 4):
            pl.semaphore_signal(bsem, inc=1,
                                device_id=(ypos(lax.rem(yc + d, 4)),),
                                device_id_type=MESH)
            pl.semaphore_signal(bsem, inc=1,
                                device_id=(zpos(lax.rem(zc + d, 4)),),
                                device_id_type=MESH)
        pl.semaphore_wait(bsem, 7)

        wins = [win0_ref, win1_ref, win2_ref]
        wouts = [wout0_ref, wout1_ref, wout2_ref]
        cur = x_ref[:, :]
        for l in range(3):
            a_ref[:, :] = jnp.dot(cur, wins[l][:, :],
                                  preferred_element_type=jnp.float32)

            rs_xa = pltpu.make_async_remote_copy(
                src_ref=a_ref.at[pl.ds((1 - xc) * 256, 256), CA],
                dst_ref=rxa_ref,
                send_sem=xa_send.at[0], recv_sem=xa_recv.at[0],
                device_id=(xpeer,), device_id_type=MESH,
            )
            rs_xa.start()

            for d in range(1, 4):
                yp = lax.rem(yc + d, 4)
                pltpu.make_async_remote_copy(
                    src_ref=a_ref.at[pl.ds(yp * 128, 128), CB],
                    dst_ref=ryb_ref.at[pl.ds((3 - d) * 128, 128), :],
                    send_sem=yb_send.at[d], recv_sem=yb_recv.at[d],
                    device_id=(ypos(yp),), device_id_type=MESH,
                ).start()

            rs_xa.wait()
            a_ref[pl.ds(xa_off, 256), CA] = (
                a_ref[pl.ds(xa_off, 256), CA] + rxa_ref[:, :])
            for d in range(1, 4):
                yp = lax.rem(yc + d, 4)
                pltpu.make_async_remote_copy(
                    src_ref=a_ref.at[pl.ds(xa_off + yp * 64, 64), CA],
                    dst_ref=rya_ref.at[pl.ds((3 - d) * 64, 64), :],
                    send_sem=ya_send.at[d], recv_sem=ya_recv.at[d],
                    device_id=(ypos(yp),), device_id_type=MESH,
                ).start()

            for d in range(1, 4):
                desc = pltpu.make_async_remote_copy(
                    src_ref=a_ref.at[pl.ds(0, 128), CB],
                    dst_ref=ryb_ref.at[pl.ds((3 - d) * 128, 128), :],
                    send_sem=yb_send.at[d], recv_sem=yb_recv.at[d],
                    device_id=(p,), device_id_type=MESH,
                )
                desc.wait_recv()
                desc.wait_send()
            a_ref[pl.ds(yb_off, 128), CB] = (
                a_ref[pl.ds(yb_off, 128), CB] + ryb_ref[pl.ds(0, 128), :]
                + ryb_ref[pl.ds(128, 128), :] + ryb_ref[pl.ds(256, 128), :])
            for d in range(1, 4):
                zp = lax.rem(zc + d, 4)
                pltpu.make_async_remote_copy(
                    src_ref=a_ref.at[pl.ds(yb_off + zp * 32, 32), CB],
                    dst_ref=rzb_ref.at[pl.ds((3 - d) * 32, 32), :],
                    send_sem=zb_send.at[d], recv_sem=zb_recv.at[d],
                    device_id=(zpos(zp),), device_id_type=MESH,
                ).start()

            for d in range(1, 4):
                desc = pltpu.make_async_remote_copy(
                    src_ref=a_ref.at[pl.ds(0, 64), CA],
                    dst_ref=rya_ref.at[pl.ds((3 - d) * 64, 64), :],
                    send_sem=ya_send.at[d], recv_sem=ya_recv.at[d],
                    device_id=(p,), device_id_type=MESH,
                )
                desc.wait_recv()
                desc.wait_send()
            a_ref[pl.ds(ya_off, 64), CA] = (
                a_ref[pl.ds(ya_off, 64), CA] + rya_ref[pl.ds(0, 64), :]
                + rya_ref[pl.ds(64, 64), :] + rya_ref[pl.ds(128, 64), :])
            for d in range(1, 4):
                zp = lax.rem(zc + d, 4)
                pltpu.make_async_remote_copy(
                    src_ref=a_ref.at[pl.ds(ya_off, 64), CA],
                    dst_ref=rza_ref.at[pl.ds((3 - d) * 64, 64), :],
                    send_sem=za_send.at[d], recv_sem=za_recv.at[d],
                    device_id=(zpos(zp),), device_id_type=MESH,
                ).start()

            for d in range(1, 4):
                desc = pltpu.make_async_remote_copy(
                    src_ref=a_ref.at[pl.ds(0, 32), CB],
                    dst_ref=rzb_ref.at[pl.ds((3 - d) * 32, 32), :],
                    send_sem=zb_send.at[d], recv_sem=zb_recv.at[d],
                    device_id=(p,), device_id_type=MESH,
                )
                desc.wait_recv()
                desc.wait_send()
            a_ref[pl.ds(zb_off, 32), CB] = (
                a_ref[pl.ds(zb_off, 32), CB] + rzb_ref[pl.ds(0, 32), :]
                + rzb_ref[pl.ds(32, 32), :] + rzb_ref[pl.ds(64, 32), :])
            ar_xb = pltpu.make_async_remote_copy(
                src_ref=a_ref.at[pl.ds(zb_off, 32), CB],
                dst_ref=rxb_ref,
                send_sem=xb_send.at[0], recv_sem=xb_recv.at[0],
                device_id=(xpeer,), device_id_type=MESH,
            )
            ar_xb.start()

            for d in range(1, 4):
                desc = pltpu.make_async_remote_copy(
                    src_ref=a_ref.at[pl.ds(0, 64), CA],
                    dst_ref=rza_ref.at[pl.ds((3 - d) * 64, 64), :],
                    send_sem=za_send.at[d], recv_sem=za_recv.at[d],
                    device_id=(p,), device_id_type=MESH,
                )
                desc.wait_recv()
                desc.wait_send()
            acc_za = (a_ref[pl.ds(ya_off, 64), CA] + rza_ref[pl.ds(0, 64), :]
                      + rza_ref[pl.ds(64, 64), :] + rza_ref[pl.ds(128, 64), :])
            h_ref[pl.ds(ya_off, 64), CA] = jnp.maximum(acc_za, 0.0)
            for d in range(1, 4):
                yp = lax.rem(yc + d, 4)
                pltpu.make_async_remote_copy(
                    src_ref=h_ref.at[pl.ds(ya_off, 64), CA],
                    dst_ref=h_ref.at[pl.ds(ya_off, 64), CA],
                    send_sem=gya_send.at[d], recv_sem=gya_recv.at[d],
                    device_id=(ypos(yp),), device_id_type=MESH,
                ).start()

            ar_xb.wait()
            acc_xb = a_ref[pl.ds(zb_off, 32), CB] + rxb_ref[:, :]
            h_ref[pl.ds(zb_off, 32), CB] = jnp.maximum(acc_xb, 0.0)
            for d in range(1, 4):
                zp = lax.rem(zc + d, 4)
                pltpu.make_async_remote_copy(
                    src_ref=h_ref.at[pl.ds(zb_off, 32), CB],
                    dst_ref=h_ref.at[pl.ds(zb_off, 32), CB],
                    send_sem=gzb_send.at[d], recv_sem=gzb_recv.at[d],
                    device_id=(zpos(zp),), device_id_type=MESH,
                ).start()

            for d in range(1, 4):
                ys = lax.rem(yc + 4 - d, 4)
                desc = pltpu.make_async_remote_copy(
                    src_ref=h_ref.at[pl.ds(ya_off, 64), CA],
                    dst_ref=h_ref.at[pl.ds(xa_off + ys * 64, 64), CA],
                    send_sem=gya_send.at[d], recv_sem=gya_recv.at[d],
                    device_id=(p,), device_id_type=MESH,
                )
                desc.wait_recv()
                desc.wait_send()
            ag_xa = pltpu.make_async_remote_copy(
                src_ref=h_ref.at[pl.ds(xa_off, 256), CA],
                dst_ref=h_ref.at[pl.ds(xa_off, 256), CA],
                send_sem=gxa_send.at[0], recv_sem=gxa_recv.at[0],
                device_id=(xpeer,), device_id_type=MESH,
            )
            ag_xa.start()

            for d in range(1, 4):
                zs = lax.rem(zc + 4 - d, 4)
                desc = pltpu.make_async_remote_copy(
                    src_ref=h_ref.at[pl.ds(zb_off, 32), CB],
                    dst_ref=h_ref.at[pl.ds(yb_off + zs * 32, 32), CB],
                    send_sem=gzb_send.at[d], recv_sem=gzb_recv.at[d],
                    device_id=(p,), device_id_type=MESH,
                )
                desc.wait_recv()
                desc.wait_send()
            for d in range(1, 4):
                yp = lax.rem(yc + d, 4)
                pltpu.make_async_remote_copy(
                    src_ref=h_ref.at[pl.ds(yb_off, 128), CB],
                    dst_ref=h_ref.at[pl.ds(yb_off, 128), CB],
                    send_sem=gyb_send.at[d], recv_sem=gyb_recv.at[d],
                    device_id=(ypos(yp),), device_id_type=MESH,
                ).start()

            ag_xa.wait_send()
            desc = pltpu.make_async_remote_copy(
                src_ref=h_ref.at[pl.ds(xa_off, 256), CA],
                dst_ref=h_ref.at[pl.ds((1 - xc) * 256, 256), CA],
                send_sem=gxa_send.at[0], recv_sem=gxa_recv.at[0],
                device_id=(p,), device_id_type=MESH,
            )
            desc.wait_recv()

            for d in range(1, 4):
                ys = lax.rem(yc + 4 - d, 4)
                desc = pltpu.make_async_remote_copy(
                    src_ref=h_ref.at[pl.ds(yb_off, 128), CB],
                    dst_ref=h_ref.at[pl.ds(ys * 128, 128), CB],
                    send_sem=gyb_send.at[d], recv_sem=gyb_recv.at[d],
                    device_id=(p,), device_id_type=MESH,
                )
                desc.wait_recv()
                desc.wait_send()

            cur = jnp.dot(h_ref[:, :], wouts[l][:, :],
                          preferred_element_type=jnp.float32)

        out_ref[:, :] = cur

    return pl.pallas_call(
        body,
        out_shape=jax.ShapeDtypeStruct((b, d_out), jnp.float32),
        in_specs=[pl.BlockSpec(memory_space=pltpu.VMEM)] * 7,
        out_specs=pl.BlockSpec(memory_space=pltpu.VMEM),
        scratch_shapes=[
            pltpu.VMEM((B, D), jnp.float32),
            pltpu.VMEM((B, D), jnp.float32),
            pltpu.VMEM((256, HC), jnp.float32),
            pltpu.VMEM((192, HC), jnp.float32),
            pltpu.VMEM((192, HC), jnp.float32),
            pltpu.VMEM((384, HC), jnp.float32),
            pltpu.VMEM((96, HC), jnp.float32),
            pltpu.VMEM((32, HC), jnp.float32),
            pltpu.SemaphoreType.DMA((1,)),
            pltpu.SemaphoreType.DMA((1,)),
            pltpu.SemaphoreType.DMA((4,)),
            pltpu.SemaphoreType.DMA((4,)),
            pltpu.SemaphoreType.DMA((4,)),
            pltpu.SemaphoreType.DMA((4,)),
            pltpu.SemaphoreType.DMA((1,)),
            pltpu.SemaphoreType.DMA((1,)),
            pltpu.SemaphoreType.DMA((4,)),
            pltpu.SemaphoreType.DMA((4,)),
            pltpu.SemaphoreType.DMA((1,)),
            pltpu.SemaphoreType.DMA((1,)),
            pltpu.SemaphoreType.DMA((4,)),
            pltpu.SemaphoreType.DMA((4,)),
            pltpu.SemaphoreType.DMA((4,)),
            pltpu.SemaphoreType.DMA((4,)),
            pltpu.SemaphoreType.DMA((4,)),
            pltpu.SemaphoreType.DMA((4,)),
            pltpu.SemaphoreType.DMA((4,)),
            pltpu.SemaphoreType.DMA((4,)),
        ],
        compiler_params=pltpu.CompilerParams(collective_id=0),
    )(x, Win0, Wout0, Win1, Wout1, Win2, Wout2)


# device time: 82944 ns/iter; 5.1974x vs baseline; 1.1011x over previous
import jax
import jax.numpy as jnp
from jax import lax
from jax.experimental import pallas as pl
from jax.experimental.pallas import tpu as pltpu

N_DEV = 32
B = 512
D = 512
HC = 256
MESH = pl.DeviceIdType.MESH


def kernel(x, Win0, Wout0, Win1, Wout1, Win2, Wout2):
    b, k_shard = x.shape
    d_out = Wout0.shape[1]

    def body(x_ref, win0_ref, wout0_ref, win1_ref, wout1_ref,
             win2_ref, wout2_ref, out_ref,
             a_ref, h_ref,
             rxa_ref, rya_ref, rza_ref, ryb_ref, rzb_ref, rxb_ref,
             xa_send, xa_recv, ya_send, ya_recv, za_send, za_recv,
             gxa_send, gxa_recv, gya_send, gya_recv,
             xb_send, xb_recv, yb_send, yb_recv, zb_send, zb_recv,
             gyb_send, gyb_recv, gzb_send, gzb_recv):
        p = lax.axis_index("i")
        t = lax.rem(p, 2)
        zc = lax.div(p, 8)
        q = lax.rem(p, 8)
        yc = lax.div(q, 2)
        yb = lax.rem(yc, 2)
        xc = t + yb - 2 * t * yb
        base8 = lax.rem(p, 8)

        xpeer = p + 1 - 2 * t

        def ypos(yp):
            ypb = lax.rem(yp, 2)
            tt = xc + ypb - 2 * xc * ypb
            return 8 * zc + 2 * yp + tt

        def zpos(zp):
            return base8 + 8 * zp

        CA = pl.ds(0, HC)
        CB = pl.ds(HC, HC)

        xa_off = xc * 256
        ya_off = xa_off + yc * 64
        za_off = ya_off + zc * 16
        yb_off = yc * 128
        zb_off = yb_off + zc * 32
        xb_off = zb_off + xc * 16

        bsem = pltpu.get_barrier_semaphore()
        pl.semaphore_signal(bsem, inc=1, device_id=(xpeer,),
                            device_id_type=MESH)
        for d in range(1, 4):
            pl.semaphore_signal(bsem, inc=1,
                                device_id=(ypos(lax.rem(yc + d, 4)),),
                                device_id_type=MESH)
            pl.semaphore_signal(bsem, inc=1,
                                device_id=(zpos(lax.rem(zc + d, 4)),),
                                device_id_type=MESH)
        pl.semaphore_wait(bsem, 7)

        wins = [win0_ref, win1_ref, win2_ref]
        wouts = [wout0_ref, wout1_ref, wout2_ref]
        cur = x_ref[:, :]
        for l in range(3):
            a_ref[:, :] = jnp.dot(cur, wins[l][:, :],
                                  preferred_element_type=jnp.float32)

            rs_xa = pltpu.make_async_remote_copy(
                src_ref=a_ref.at[pl.ds((1 - xc) * 256, 256), CA],
                dst_ref=rxa_ref,
                send_sem=xa_send.at[0], recv_sem=xa_recv.at[0],
                device_id=(xpeer,), device_id_type=MESH,
            )
            rs_xa.start()

            for d in range(1, 4):
                yp = lax.rem(yc + d, 4)
                pltpu.make_async_remote_copy(
                    src_ref=a_ref.at[pl.ds(yp * 128, 128), CB],
                    dst_ref=ryb_ref.at[pl.ds((3 - d) * 128, 128), :],
                    send_sem=yb_send.at[d], recv_sem=yb_recv.at[d],
                    device_id=(ypos(yp),), device_id_type=MESH,
                ).start()

            rs_xa.wait()
            a_ref[pl.ds(xa_off, 256), CA] = (
                a_ref[pl.ds(xa_off, 256), CA] + rxa_ref[:, :])
            for d in range(1, 4):
                yp = lax.rem(yc + d, 4)
                pltpu.make_async_remote_copy(
                    src_ref=a_ref.at[pl.ds(xa_off + yp * 64, 64), CA],
                    dst_ref=rya_ref.at[pl.ds((3 - d) * 64, 64), :],
                    send_sem=ya_send.at[d], recv_sem=ya_recv.at[d],
                    device_id=(ypos(yp),), device_id_type=MESH,
                ).start()

            for d in range(1, 4):
                desc = pltpu.make_async_remote_copy(
                    src_ref=a_ref.at[pl.ds(0, 128), CB],
                    dst_ref=ryb_ref.at[pl.ds((3 - d) * 128, 128), :],
                    send_sem=yb_send.at[d], recv_sem=yb_recv.at[d],
                    device_id=(p,), device_id_type=MESH,
                )
                desc.wait_recv()
                desc.wait_send()
            a_ref[pl.ds(yb_off, 128), CB] = (
                a_ref[pl.ds(yb_off, 128), CB] + ryb_ref[pl.ds(0, 128), :]
                + ryb_ref[pl.ds(128, 128), :] + ryb_ref[pl.ds(256, 128), :])
            for d in range(1, 4):
                zp = lax.rem(zc + d, 4)
                pltpu.make_async_remote_copy(
                    src_ref=a_ref.at[pl.ds(yb_off + zp * 32, 32), CB],
                    dst_ref=rzb_ref.at[pl.ds((3 - d) * 32, 32), :],
                    send_sem=zb_send.at[d], recv_sem=zb_recv.at[d],
                    device_id=(zpos(zp),), device_id_type=MESH,
                ).start()

            for d in range(1, 4):
                desc = pltpu.make_async_remote_copy(
                    src_ref=a_ref.at[pl.ds(0, 64), CA],
                    dst_ref=rya_ref.at[pl.ds((3 - d) * 64, 64), :],
                    send_sem=ya_send.at[d], recv_sem=ya_recv.at[d],
                    device_id=(p,), device_id_type=MESH,
                )
                desc.wait_recv()
                desc.wait_send()
            a_ref[pl.ds(ya_off, 64), CA] = (
                a_ref[pl.ds(ya_off, 64), CA] + rya_ref[pl.ds(0, 64), :]
                + rya_ref[pl.ds(64, 64), :] + rya_ref[pl.ds(128, 64), :])
            for d in range(1, 4):
                zp = lax.rem(zc + d, 4)
                pltpu.make_async_remote_copy(
                    src_ref=a_ref.at[pl.ds(ya_off, 64), CA],
                    dst_ref=rza_ref.at[pl.ds((3 - d) * 64, 64), :],
                    send_sem=za_send.at[d], recv_sem=za_recv.at[d],
                    device_id=(zpos(zp),), device_id_type=MESH,
                ).start()

            for d in range(1, 4):
                desc = pltpu.make_async_remote_copy(
                    src_ref=a_ref.at[pl.ds(0, 32), CB],
                    dst_ref=rzb_ref.at[pl.ds((3 - d) * 32, 32), :],
                    send_sem=zb_send.at[d], recv_sem=zb_recv.at[d],
                    device_id=(p,), device_id_type=MESH,
                )
                desc.wait_recv()
                desc.wait_send()
            a_ref[pl.ds(zb_off, 32), CB] = (
                a_ref[pl.ds(zb_off, 32), CB] + rzb_ref[pl.ds(0, 32), :]
                + rzb_ref[pl.ds(32, 32), :] + rzb_ref[pl.ds(64, 32), :])
            ar_xb = pltpu.make_async_remote_copy(
                src_ref=a_ref.at[pl.ds(zb_off, 32), CB],
                dst_ref=rxb_ref,
                send_sem=xb_send.at[0], recv_sem=xb_recv.at[0],
                device_id=(xpeer,), device_id_type=MESH,
            )
            ar_xb.start()

            for d in range(1, 4):
                desc = pltpu.make_async_remote_copy(
                    src_ref=a_ref.at[pl.ds(0, 64), CA],
                    dst_ref=rza_ref.at[pl.ds((3 - d) * 64, 64), :],
                    send_sem=za_send.at[d], recv_sem=za_recv.at[d],
                    device_id=(p,), device_id_type=MESH,
                )
                desc.wait_recv()
                desc.wait_send()
            acc_za = (a_ref[pl.ds(ya_off, 64), CA] + rza_ref[pl.ds(0, 64), :]
                      + rza_ref[pl.ds(64, 64), :] + rza_ref[pl.ds(128, 64), :])
            h_ref[pl.ds(ya_off, 64), CA] = jnp.maximum(
                acc_za, 0.0).astype(jnp.bfloat16)
            for d in range(1, 4):
                yp = lax.rem(yc + d, 4)
                pltpu.make_async_remote_copy(
                    src_ref=h_ref.at[pl.ds(ya_off, 64), CA],
                    dst_ref=h_ref.at[pl.ds(ya_off, 64), CA],
                    send_sem=gya_send.at[d], recv_sem=gya_recv.at[d],
                    device_id=(ypos(yp),), device_id_type=MESH,
                ).start()

            ar_xb.wait()
            acc_xb = a_ref[pl.ds(zb_off, 32), CB] + rxb_ref[:, :]
            h_ref[pl.ds(zb_off, 32), CB] = jnp.maximum(
                acc_xb, 0.0).astype(jnp.bfloat16)
            for d in range(1, 4):
                zp = lax.rem(zc + d, 4)
                pltpu.make_async_remote_copy(
                    src_ref=h_ref.at[pl.ds(zb_off, 32), CB],
                    dst_ref=h_ref.at[pl.ds(zb_off, 32), CB],
                    send_sem=gzb_send.at[d], recv_sem=gzb_recv.at[d],
                    device_id=(zpos(zp),), device_id_type=MESH,
                ).start()

            for d in range(1, 4):
                ys = lax.rem(yc + 4 - d, 4)
                desc = pltpu.make_async_remote_copy(
                    src_ref=h_ref.at[pl.ds(ya_off, 64), CA],
                    dst_ref=h_ref.at[pl.ds(xa_off + ys * 64, 64), CA],
                    send_sem=gya_send.at[d], recv_sem=gya_recv.at[d],
                    device_id=(p,), device_id_type=MESH,
                )
                desc.wait_recv()
                desc.wait_send()
            ag_xa = pltpu.make_async_remote_copy(
                src_ref=h_ref.at[pl.ds(xa_off, 256), CA],
                dst_ref=h_ref.at[pl.ds(xa_off, 256), CA],
                send_sem=gxa_send.at[0], recv_sem=gxa_recv.at[0],
                device_id=(xpeer,), device_id_type=MESH,
            )
            ag_xa.start()

            for d in range(1, 4):
                zs = lax.rem(zc + 4 - d, 4)
                desc = pltpu.make_async_remote_copy(
                    src_ref=h_ref.at[pl.ds(zb_off, 32), CB],
                    dst_ref=h_ref.at[pl.ds(yb_off + zs * 32, 32), CB],
                    send_sem=gzb_send.at[d], recv_sem=gzb_recv.at[d],
                    device_id=(p,), device_id_type=MESH,
                )
                desc.wait_recv()
                desc.wait_send()
            for d in range(1, 4):
                yp = lax.rem(yc + d, 4)
                pltpu.make_async_remote_copy(
                    src_ref=h_ref.at[pl.ds(yb_off, 128), CB],
                    dst_ref=h_ref.at[pl.ds(yb_off, 128), CB],
                    send_sem=gyb_send.at[d], recv_sem=gyb_recv.at[d],
                    device_id=(ypos(yp),), device_id_type=MESH,
                ).start()

            ag_xa.wait_send()
            desc = pltpu.make_async_remote_copy(
                src_ref=h_ref.at[pl.ds(xa_off, 256), CA],
                dst_ref=h_ref.at[pl.ds((1 - xc) * 256, 256), CA],
                send_sem=gxa_send.at[0], recv_sem=gxa_recv.at[0],
                device_id=(p,), device_id_type=MESH,
            )
            desc.wait_recv()

            for d in range(1, 4):
                ys = lax.rem(yc + 4 - d, 4)
                desc = pltpu.make_async_remote_copy(
                    src_ref=h_ref.at[pl.ds(yb_off, 128), CB],
                    dst_ref=h_ref.at[pl.ds(ys * 128, 128), CB],
                    send_sem=gyb_send.at[d], recv_sem=gyb_recv.at[d],
                    device_id=(p,), device_id_type=MESH,
                )
                desc.wait_recv()
                desc.wait_send()

            cur = jnp.dot(h_ref[:, :], wouts[l][:, :].astype(jnp.bfloat16),
                          preferred_element_type=jnp.float32)

        out_ref[:, :] = cur

    return pl.pallas_call(
        body,
        out_shape=jax.ShapeDtypeStruct((b, d_out), jnp.float32),
        in_specs=[pl.BlockSpec(memory_space=pltpu.VMEM)] * 7,
        out_specs=pl.BlockSpec(memory_space=pltpu.VMEM),
        scratch_shapes=[
            pltpu.VMEM((B, D), jnp.float32),
            pltpu.VMEM((B, D), jnp.bfloat16),
            pltpu.VMEM((256, HC), jnp.float32),
            pltpu.VMEM((192, HC), jnp.float32),
            pltpu.VMEM((192, HC), jnp.float32),
            pltpu.VMEM((384, HC), jnp.float32),
            pltpu.VMEM((96, HC), jnp.float32),
            pltpu.VMEM((32, HC), jnp.float32),
            pltpu.SemaphoreType.DMA((1,)),
            pltpu.SemaphoreType.DMA((1,)),
            pltpu.SemaphoreType.DMA((4,)),
            pltpu.SemaphoreType.DMA((4,)),
            pltpu.SemaphoreType.DMA((4,)),
            pltpu.SemaphoreType.DMA((4,)),
            pltpu.SemaphoreType.DMA((1,)),
            pltpu.SemaphoreType.DMA((1,)),
            pltpu.SemaphoreType.DMA((4,)),
            pltpu.SemaphoreType.DMA((4,)),
            pltpu.SemaphoreType.DMA((1,)),
            pltpu.SemaphoreType.DMA((1,)),
            pltpu.SemaphoreType.DMA((4,)),
            pltpu.SemaphoreType.DMA((4,)),
            pltpu.SemaphoreType.DMA((4,)),
            pltpu.SemaphoreType.DMA((4,)),
            pltpu.SemaphoreType.DMA((4,)),
            pltpu.SemaphoreType.DMA((4,)),
            pltpu.SemaphoreType.DMA((4,)),
            pltpu.SemaphoreType.DMA((4,)),
        ],
        compiler_params=pltpu.CompilerParams(collective_id=0),
    )(x, Win0, Wout0, Win1, Wout1, Win2, Wout2)
